# baseline (speedup 1.0000x reference)
import functools
import math

import jax
import jax.numpy as jnp
from jax import lax
from jax.experimental import pallas as pl
from jax.experimental.pallas import tpu as pltpu
from jax.experimental.pallas import tpu_sc as plsc

CB_SIZE = 8192
CB_DIM = 64
LMBDA = 0.5
B = 16384
BM = 256

_LOG2 = math.log(2.0)


def _tc_body(x_ref, cbt_ref, logits_ref, idx_ref, bits_ref):
    i = pl.program_id(0)

    logits = logits_ref[...]
    m = jnp.max(logits, axis=-1, keepdims=True)
    shifted = logits - m
    lse = jnp.log(jnp.sum(jnp.exp(shifted), axis=-1, keepdims=True))
    log2_pmf = -(shifted - lse) / _LOG2

    xb = x_ref[...]
    cbt = cbt_ref[...]

    x2 = jnp.sum(xb * xb, axis=1, keepdims=True)
    c2 = jnp.sum(cbt * cbt, axis=0, keepdims=True)
    cross = jnp.dot(xb, cbt, preferred_element_type=jnp.float32)

    t = jnp.maximum(x2 + c2 - 2.0 * cross, 0.0)
    s = jnp.where(t == 0.0, 0.0, t * lax.rsqrt(t))
    dist = s + log2_pmf / LMBDA

    minval = jnp.min(dist, axis=1, keepdims=True)
    cols = lax.broadcasted_iota(jnp.int32, dist.shape, 1)
    idx = jnp.min(jnp.where(dist == minval, cols, CB_SIZE), axis=1)

    sel = cols == idx[:, None]
    lp = jnp.min(jnp.where(sel, log2_pmf, jnp.inf), axis=1)

    idx_ref[...] = idx[:, None]

    @pl.when(i == 0)
    def _init():
        bits_ref[...] = jnp.zeros_like(bits_ref)

    bits_ref[...] += jnp.sum(lp).reshape(1, 1)


def _tc_quant(x, cbt, logits):
    grid = (B // BM,)
    return pl.pallas_call(
        _tc_body,
        grid=grid,
        in_specs=[
            pl.BlockSpec((BM, CB_DIM), lambda i: (i, 0)),
            pl.BlockSpec((CB_DIM, CB_SIZE), lambda i: (0, 0)),
            pl.BlockSpec((1, CB_SIZE), lambda i: (0, 0)),
        ],
        out_specs=[
            pl.BlockSpec((BM, 1), lambda i: (i, 0)),
            pl.BlockSpec((1, 1), lambda i: (0, 0)),
        ],
        out_shape=[
            jax.ShapeDtypeStruct((B, 1), jnp.int32),
            jax.ShapeDtypeStruct((1, 1), jnp.float32),
        ],
    )(x, cbt, logits)


_SC_CHUNK = 128


def _make_sc_gather():
    info = plsc.get_sparse_core_info()
    nw = info.num_cores * info.num_subcores
    b_per_w = B // nw
    nchunk = b_per_w // _SC_CHUNK
    mesh = plsc.VectorSubcoreMesh(core_axis_name="c", subcore_axis_name="s")

    @functools.partial(
        pl.kernel,
        mesh=mesh,
        out_type=jax.ShapeDtypeStruct((B, CB_DIM), jnp.float32),
        scratch_types=[
            pltpu.VMEM((nchunk, _SC_CHUNK), jnp.int32),
            pltpu.VMEM((b_per_w, CB_DIM), jnp.float32),
            pltpu.SemaphoreType.DMA,
        ],
        compiler_params=pltpu.CompilerParams(use_tc_tiling_on_sc=False),
    )
    def sc_gather(cb_hbm, idx_hbm, out_hbm, idx_v, rows_v, sem):
        wid = lax.axis_index("s") * info.num_cores + lax.axis_index("c")
        base = wid * b_per_w
        pltpu.sync_copy(idx_hbm.at[pl.ds(wid * nchunk, nchunk)], idx_v)
        for j in range(nchunk):
            pltpu.async_copy(
                cb_hbm.at[idx_v.at[j]],
                rows_v.at[pl.ds(j * _SC_CHUNK, _SC_CHUNK)],
                sem,
            ).wait()
        pltpu.sync_copy(rows_v, out_hbm.at[pl.ds(base, b_per_w)])

    return sc_gather


def kernel(x, codebook, logits):
    cb2d = codebook[0]
    cbt = cb2d.T
    idx2d, bits = _tc_quant(x, cbt, logits)
    idx_rows = idx2d.reshape(B // _SC_CHUNK, _SC_CHUNK)
    x_hat = _make_sc_gather()(cb2d, idx_rows)
    return (x_hat, bits.reshape(()), idx2d)


# speedup vs baseline: 1.1495x; 1.1495x over previous
import functools
import math

import jax
import jax.numpy as jnp
from jax import lax
from jax.experimental import pallas as pl
from jax.experimental.pallas import tpu as pltpu
from jax.experimental.pallas import tpu_sc as plsc

CB_SIZE = 8192
CB_DIM = 64
LMBDA = 0.5
B = 16384
BM = 512

_LOG2 = math.log(2.0)


def _tc_body(x_ref, cbt_ref, logits_ref, idx_ref, bits_ref):
    i = pl.program_id(0)

    logits = logits_ref[...]
    m = jnp.max(logits, axis=-1, keepdims=True)
    shifted = logits - m
    lse = jnp.log(jnp.sum(jnp.exp(shifted), axis=-1, keepdims=True))
    log2_pmf = -(shifted - lse) / _LOG2

    xb = x_ref[...]
    cbt = cbt_ref[...]

    x2 = jnp.sum(xb * xb, axis=1, keepdims=True)
    c2 = jnp.sum(cbt * cbt, axis=0, keepdims=True)
    cross = jnp.dot(xb, cbt, preferred_element_type=jnp.float32)

    t = jnp.maximum(x2 + c2 - 2.0 * cross, 0.0)
    s = jnp.where(t == 0.0, 0.0, t * lax.rsqrt(t))
    dist = s + log2_pmf / LMBDA

    idx = jnp.argmin(dist, axis=1).astype(jnp.int32)
    cols = lax.broadcasted_iota(jnp.int32, dist.shape, 1)

    sel = cols == idx[:, None]
    lp = jnp.min(jnp.where(sel, log2_pmf, jnp.inf), axis=1)

    idx_ref[...] = idx[:, None]

    @pl.when(i == 0)
    def _init():
        bits_ref[...] = jnp.zeros_like(bits_ref)

    bits_ref[...] += jnp.sum(lp).reshape(1, 1)


def _tc_quant(x, cbt, logits):
    grid = (B // BM,)
    return pl.pallas_call(
        _tc_body,
        grid=grid,
        in_specs=[
            pl.BlockSpec((BM, CB_DIM), lambda i: (i, 0)),
            pl.BlockSpec((CB_DIM, CB_SIZE), lambda i: (0, 0)),
            pl.BlockSpec((1, CB_SIZE), lambda i: (0, 0)),
        ],
        out_specs=[
            pl.BlockSpec((BM, 1), lambda i: (i, 0)),
            pl.BlockSpec((1, 1), lambda i: (0, 0)),
        ],
        out_shape=[
            jax.ShapeDtypeStruct((B, 1), jnp.int32),
            jax.ShapeDtypeStruct((1, 1), jnp.float32),
        ],
    )(x, cbt, logits)


_SC_CHUNK = 128


def _make_sc_gather():
    info = plsc.get_sparse_core_info()
    nw = info.num_cores * info.num_subcores
    b_per_w = B // nw
    nchunk = b_per_w // _SC_CHUNK
    mesh = plsc.VectorSubcoreMesh(core_axis_name="c", subcore_axis_name="s")

    @functools.partial(
        pl.kernel,
        mesh=mesh,
        out_type=jax.ShapeDtypeStruct((B, CB_DIM), jnp.float32),
        scratch_types=[
            pltpu.VMEM((nchunk, _SC_CHUNK), jnp.int32),
            pltpu.VMEM((b_per_w, CB_DIM), jnp.float32),
            pltpu.SemaphoreType.DMA,
        ],
        compiler_params=pltpu.CompilerParams(use_tc_tiling_on_sc=False),
    )
    def sc_gather(cb_hbm, idx_hbm, out_hbm, idx_v, rows_v, sem):
        wid = lax.axis_index("s") * info.num_cores + lax.axis_index("c")
        base = wid * b_per_w
        pltpu.sync_copy(idx_hbm.at[pl.ds(wid * nchunk, nchunk)], idx_v)
        for j in range(nchunk):
            pltpu.async_copy(
                cb_hbm.at[idx_v.at[j]],
                rows_v.at[pl.ds(j * _SC_CHUNK, _SC_CHUNK)],
                sem,
            ).wait()
        pltpu.sync_copy(rows_v, out_hbm.at[pl.ds(base, b_per_w)])

    return sc_gather


def kernel(x, codebook, logits):
    cb2d = codebook[0]
    cbt = cb2d.T
    idx2d, bits = _tc_quant(x, cbt, logits)
    idx_rows = idx2d.reshape(B // _SC_CHUNK, _SC_CHUNK)
    x_hat = _make_sc_gather()(cb2d, idx_rows)
    return (x_hat, bits.reshape(()), idx2d)


# speedup vs baseline: 1.2196x; 1.0610x over previous
import functools
import math

import jax
import jax.numpy as jnp
from jax import lax
from jax.experimental import pallas as pl
from jax.experimental.pallas import tpu as pltpu
from jax.experimental.pallas import tpu_sc as plsc

CB_SIZE = 8192
CB_DIM = 64
LMBDA = 0.5
B = 16384
BM = 1024

_LOG2 = math.log(2.0)


def _tc_body(x_ref, cbt_ref, logits_ref, idx_ref, bits_ref):
    i = pl.program_id(0)

    logits = logits_ref[...]
    m = jnp.max(logits, axis=-1, keepdims=True)
    shifted = logits - m
    lse = jnp.log(jnp.sum(jnp.exp(shifted), axis=-1, keepdims=True))
    log2_pmf = -(shifted - lse) / _LOG2

    xb = x_ref[...]
    cbt = cbt_ref[...]

    x2 = jnp.sum(xb * xb, axis=1, keepdims=True)
    c2 = jnp.sum(cbt * cbt, axis=0, keepdims=True)
    cross = jnp.dot(xb, cbt, preferred_element_type=jnp.float32)

    t = jnp.maximum(x2 + c2 - 2.0 * cross, 0.0)
    s = jnp.where(t == 0.0, 0.0, t * lax.rsqrt(t))
    dist = s + log2_pmf / LMBDA

    idx = jnp.argmin(dist, axis=1).astype(jnp.int32)
    cols = lax.broadcasted_iota(jnp.int32, dist.shape, 1)

    sel = cols == idx[:, None]
    lp = jnp.min(jnp.where(sel, log2_pmf, jnp.inf), axis=1)

    idx_ref[...] = idx[:, None]

    @pl.when(i == 0)
    def _init():
        bits_ref[...] = jnp.zeros_like(bits_ref)

    bits_ref[...] += jnp.sum(lp).reshape(1, 1)


def _tc_quant(x, cbt, logits):
    grid = (B // BM,)
    return pl.pallas_call(
        _tc_body,
        grid=grid,
        in_specs=[
            pl.BlockSpec((BM, CB_DIM), lambda i: (i, 0)),
            pl.BlockSpec((CB_DIM, CB_SIZE), lambda i: (0, 0)),
            pl.BlockSpec((1, CB_SIZE), lambda i: (0, 0)),
        ],
        out_specs=[
            pl.BlockSpec((BM, 1), lambda i: (i, 0)),
            pl.BlockSpec((1, 1), lambda i: (0, 0)),
        ],
        out_shape=[
            jax.ShapeDtypeStruct((B, 1), jnp.int32),
            jax.ShapeDtypeStruct((1, 1), jnp.float32),
        ],
    )(x, cbt, logits)


_SC_CHUNK = 128


def _make_sc_gather():
    info = plsc.get_sparse_core_info()
    nw = info.num_cores * info.num_subcores
    b_per_w = B // nw
    nchunk = b_per_w // _SC_CHUNK
    mesh = plsc.VectorSubcoreMesh(core_axis_name="c", subcore_axis_name="s")

    @functools.partial(
        pl.kernel,
        mesh=mesh,
        out_type=jax.ShapeDtypeStruct((B, CB_DIM), jnp.float32),
        scratch_types=[
            pltpu.VMEM((nchunk, _SC_CHUNK), jnp.int32),
            pltpu.VMEM((b_per_w, CB_DIM), jnp.float32),
            pltpu.SemaphoreType.DMA,
        ],
        compiler_params=pltpu.CompilerParams(use_tc_tiling_on_sc=False),
    )
    def sc_gather(cb_hbm, idx_hbm, out_hbm, idx_v, rows_v, sem):
        wid = lax.axis_index("s") * info.num_cores + lax.axis_index("c")
        base = wid * b_per_w
        pltpu.sync_copy(idx_hbm.at[pl.ds(wid * nchunk, nchunk)], idx_v)
        for j in range(nchunk):
            pltpu.async_copy(
                cb_hbm.at[idx_v.at[j]],
                rows_v.at[pl.ds(j * _SC_CHUNK, _SC_CHUNK)],
                sem,
            ).wait()
        pltpu.sync_copy(rows_v, out_hbm.at[pl.ds(base, b_per_w)])

    return sc_gather


def kernel(x, codebook, logits):
    cb2d = codebook[0]
    cbt = cb2d.T
    idx2d, bits = _tc_quant(x, cbt, logits)
    idx_rows = idx2d.reshape(B // _SC_CHUNK, _SC_CHUNK)
    x_hat = _make_sc_gather()(cb2d, idx_rows)
    return (x_hat, bits.reshape(()), idx2d)
